# baseline (speedup 1.0000x reference)
import functools

import jax
import jax.numpy as jnp
from jax import lax
from jax.experimental import pallas as pl
from jax.experimental.pallas import tpu as pltpu
from jax.experimental.pallas import tpu_sc as plsc

_LANES = 16
_MAX_IDX = 128


@functools.lru_cache(maxsize=None)
def _make_pool(B, H, V, F):
    info = plsc.get_sparse_core_info()
    NC, NS = info.num_cores, info.num_subcores
    NW = NC * NS
    assert B % NW == 0
    rows_per_w = B // NW
    C = 32
    assert rows_per_w % C == 0
    nchunk = rows_per_w // C
    idx_per_chunk = C * H
    assert idx_per_chunk % _MAX_IDX == 0
    ng = idx_per_chunk // _MAX_IDX
    nvec = F // _LANES

    mesh = plsc.VectorSubcoreMesh(core_axis_name="c", subcore_axis_name="s")

    @functools.partial(
        pl.kernel,
        mesh=mesh,
        compiler_params=pltpu.CompilerParams(use_tc_tiling_on_sc=False),
        out_type=jax.ShapeDtypeStruct((B, F), jnp.float32),
        scratch_types=[
            pltpu.VMEM((idx_per_chunk,), jnp.int32),
            pltpu.VMEM((idx_per_chunk,), jnp.int32),
            pltpu.VMEM((idx_per_chunk, F), jnp.float32),
            pltpu.VMEM((idx_per_chunk, F), jnp.float32),
            pltpu.VMEM((C, F), jnp.float32),
            pltpu.VMEM((C, F), jnp.float32),
            pltpu.SemaphoreType.DMA,
            pltpu.SemaphoreType.DMA,
            pltpu.SemaphoreType.DMA,
            pltpu.SemaphoreType.DMA,
        ],
    )
    def pool(xf, table, out, idx0, idx1, rows0, rows1, outc0, outc1,
             sem0, sem1, osem0, osem1):
        wid = lax.axis_index("s") * NC + lax.axis_index("c")
        idxs, rows, outcs = [idx0, idx1], [rows0, rows1], [outc0, outc1]
        sems, osems = [sem0, sem1], [osem0, osem1]

        def start(c):
            b = c % 2
            row0 = wid * rows_per_w + c * C
            pltpu.sync_copy(xf.at[pl.ds(row0 * H, idx_per_chunk)], idxs[b])
            for g in range(idx_per_chunk // _LANES):
                v = idxs[b][pl.ds(g * _LANES, _LANES)]
                j = ((v & jnp.int32(~(_WB - 1)))
                     + ((v & jnp.int32(_HB - 1)) << 1)
                     + ((v >> (_HB.bit_length() - 1)) & 1))
                idxs[b][pl.ds(g * _LANES, _LANES)] = j
            return [pltpu.async_copy(
                table.at[idxs[b].at[pl.ds(g * _MAX_IDX, _MAX_IDX)]],
                rows[b].at[pl.ds(g * _MAX_IDX, _MAX_IDX), :],
                sems[b],
            ) for g in range(ng)]

        def finish(c, copies, ocopy):
            b = c % 2
            for cp in copies:
                cp.wait()
            if ocopy is not None:
                ocopy.wait()
            rows_v, outc_v = rows[b], outcs[b]

            def row_body(r, _):
                for j in range(nvec):
                    a = rows_v[r * H, pl.ds(j * _LANES, _LANES)]
                    for t in range(1, H):
                        a = a + rows_v[r * H + t, pl.ds(j * _LANES, _LANES)]
                    outc_v[r, pl.ds(j * _LANES, _LANES)] = a
                return 0

            lax.fori_loop(0, C, row_body, 0)
            row0 = wid * rows_per_w + c * C
            return pltpu.async_copy(outc_v, out.at[pl.ds(row0, C), :],
                                    osems[b])

        copies = [start(0), None]
        ocopies = [None, None]
        for c in range(nchunk):
            if c + 1 < nchunk:
                copies[(c + 1) % 2] = start(c + 1)
            ocopies[c % 2] = finish(c, copies[c % 2], ocopies[c % 2])
        ocopies[(nchunk - 2) % 2].wait()
        ocopies[(nchunk - 1) % 2].wait()

    return pool


_WB = 16384
_HB = _WB // 2


def _xpose_body(t_ref, o_ref, *, F):
    o_ref[:, 0:F] = t_ref[:, 0:_HB].T
    o_ref[:, F:2 * F] = t_ref[:, _HB:_WB].T


@functools.lru_cache(maxsize=None)
def _make_xpose(V, F):
    nblk = pl.cdiv(V, _WB)
    return pl.pallas_call(
        functools.partial(_xpose_body, F=F),
        grid=(nblk,),
        in_specs=[pl.BlockSpec((F, _WB), lambda i: (0, i))],
        out_specs=pl.BlockSpec((_HB, 2 * F), lambda i: (i, 0)),
        out_shape=jax.ShapeDtypeStruct((nblk * _HB, 2 * F), jnp.float32),
        compiler_params=pltpu.CompilerParams(
            dimension_semantics=("parallel",)),
    )


def _dense_body(p_ref, b_ref, g_ref, be_ref, mm_ref, mv_ref, w_ref, o_ref, *, H):
    h = p_ref[:] * (1.0 / H)
    y = jnp.dot(h, w_ref[:], preferred_element_type=jnp.float32,
                precision=lax.Precision.HIGHEST)
    y = y + b_ref[:]
    inv = g_ref[:] * lax.rsqrt(mv_ref[:] + 1e-3)
    y = (y - mm_ref[:]) * inv + be_ref[:]
    d = jnp.sqrt(jnp.maximum(jnp.sum(y * y, axis=-1, keepdims=True), 1e-12))
    o_ref[:] = y / d


@functools.lru_cache(maxsize=None)
def _make_dense(B, H, F):
    BLK = 2048
    assert B % BLK == 0
    vec_spec = pl.BlockSpec((1, F), lambda i: (0, 0))
    return pl.pallas_call(
        functools.partial(_dense_body, H=H),
        grid=(B // BLK,),
        in_specs=[pl.BlockSpec((BLK, F), lambda i: (i, 0))] + [vec_spec] * 5
        + [pl.BlockSpec((F, F), lambda i: (0, 0))],
        out_specs=pl.BlockSpec((BLK, F), lambda i: (i, 0)),
        out_shape=jax.ShapeDtypeStruct((B, F), jnp.float32),
    )


def kernel(x, table, W, b, gamma, beta, moving_mean, moving_var):
    B, H = x.shape
    V, F = table.shape
    xf = jnp.reshape(x.astype(jnp.int32), (B * H,))
    packed = _make_xpose(V, F)(table.T)
    table_rm = jnp.reshape(packed, (packed.shape[0] * 2, F))
    pooled = _make_pool(B, H, V, F)(xf, table_rm)
    dense = _make_dense(B, H, F)
    row = lambda v: jnp.reshape(v, (1, F))
    return dense(pooled, row(b), row(gamma), row(beta), row(moving_mean),
                 row(moving_var), W)


# speedup vs baseline: 1.0466x; 1.0466x over previous
import functools

import jax
import jax.numpy as jnp
from jax import lax
from jax.experimental import pallas as pl
from jax.experimental.pallas import tpu as pltpu
from jax.experimental.pallas import tpu_sc as plsc

_LANES = 16
_MAX_IDX = 128


@functools.lru_cache(maxsize=None)
def _make_pool(B, H, V, F):
    info = plsc.get_sparse_core_info()
    NC, NS = info.num_cores, info.num_subcores
    NW = NC * NS
    assert B % NW == 0
    rows_per_w = B // NW
    C = 32
    assert rows_per_w % C == 0
    nchunk = rows_per_w // C
    idx_per_chunk = C * H
    assert idx_per_chunk % _MAX_IDX == 0
    ng = idx_per_chunk // _MAX_IDX
    nvec = F // _LANES

    mesh = plsc.VectorSubcoreMesh(core_axis_name="c", subcore_axis_name="s")

    @functools.partial(
        pl.kernel,
        mesh=mesh,
        compiler_params=pltpu.CompilerParams(use_tc_tiling_on_sc=False),
        out_type=jax.ShapeDtypeStruct((B, F), jnp.float32),
        scratch_types=[
            pltpu.VMEM((idx_per_chunk,), jnp.int32),
            pltpu.VMEM((idx_per_chunk,), jnp.int32),
            pltpu.VMEM((idx_per_chunk,), jnp.int32),
            pltpu.VMEM((idx_per_chunk, F), jnp.float32),
            pltpu.VMEM((idx_per_chunk, F), jnp.float32),
            pltpu.VMEM((idx_per_chunk, F), jnp.float32),
            pltpu.VMEM((C, F), jnp.float32),
            pltpu.VMEM((C, F), jnp.float32),
            pltpu.SemaphoreType.DMA,
            pltpu.SemaphoreType.DMA,
            pltpu.SemaphoreType.DMA,
            pltpu.SemaphoreType.DMA,
            pltpu.SemaphoreType.DMA,
        ],
    )
    def pool(xf, table, out, idx0, idx1, idx2, rows0, rows1, rows2,
             outc0, outc1, sem0, sem1, sem2, osem0, osem1):
        wid = lax.axis_index("s") * NC + lax.axis_index("c")
        idxs, rows, outcs = [idx0, idx1, idx2], [rows0, rows1, rows2], [outc0, outc1]
        sems, osems = [sem0, sem1, sem2], [osem0, osem1]
        NB = 3

        def start(c):
            b = c % NB
            row0 = wid * rows_per_w + c * C
            pltpu.sync_copy(xf.at[pl.ds(row0 * H, idx_per_chunk)], idxs[b])
            for g in range(idx_per_chunk // _LANES):
                v = idxs[b][pl.ds(g * _LANES, _LANES)]
                j = ((v & jnp.int32(~(_WB - 1)))
                     + ((v & jnp.int32(_HB - 1)) << 1)
                     + ((v >> (_HB.bit_length() - 1)) & 1))
                idxs[b][pl.ds(g * _LANES, _LANES)] = j
            return [pltpu.async_copy(
                table.at[idxs[b].at[pl.ds(g * _MAX_IDX, _MAX_IDX)]],
                rows[b].at[pl.ds(g * _MAX_IDX, _MAX_IDX), :],
                sems[b],
            ) for g in range(ng)]

        def finish(c, copies, ocopy):
            b = c % NB
            for cp in copies:
                cp.wait()
            if ocopy is not None:
                ocopy.wait()
            rows_v, outc_v = rows[b], outcs[c % 2]

            def row_body(r, _):
                for j in range(nvec):
                    a = rows_v[r * H, pl.ds(j * _LANES, _LANES)]
                    for t in range(1, H):
                        a = a + rows_v[r * H + t, pl.ds(j * _LANES, _LANES)]
                    outc_v[r, pl.ds(j * _LANES, _LANES)] = a
                return 0

            lax.fori_loop(0, C, row_body, 0)
            row0 = wid * rows_per_w + c * C
            return pltpu.async_copy(outc_v, out.at[pl.ds(row0, C), :],
                                    osems[c % 2])

        copies = [start(0), start(1), None]
        ocopies = [None, None]
        for c in range(nchunk):
            if c + 2 < nchunk:
                copies[(c + 2) % NB] = start(c + 2)
            ocopies[c % 2] = finish(c, copies[c % NB], ocopies[c % 2])
        ocopies[(nchunk - 2) % 2].wait()
        ocopies[(nchunk - 1) % 2].wait()

    return pool


_WB = 32768
_HB = _WB // 2


def _xpose_body(t_ref, o_ref, *, F):
    o_ref[:, 0:F] = t_ref[:, 0:_HB].T
    o_ref[:, F:2 * F] = t_ref[:, _HB:_WB].T


@functools.lru_cache(maxsize=None)
def _make_xpose(V, F):
    nblk = pl.cdiv(V, _WB)
    return pl.pallas_call(
        functools.partial(_xpose_body, F=F),
        grid=(nblk,),
        in_specs=[pl.BlockSpec((F, _WB), lambda i: (0, i))],
        out_specs=pl.BlockSpec((_HB, 2 * F), lambda i: (i, 0)),
        out_shape=jax.ShapeDtypeStruct((nblk * _HB, 2 * F), jnp.float32),
        compiler_params=pltpu.CompilerParams(
            dimension_semantics=("parallel",)),
    )


def _dense_body(p_ref, b_ref, g_ref, be_ref, mm_ref, mv_ref, w_ref, o_ref, *, H):
    h = p_ref[:] * (1.0 / H)
    y = jnp.dot(h, w_ref[:], preferred_element_type=jnp.float32,
                precision=lax.Precision.HIGHEST)
    y = y + b_ref[:]
    inv = g_ref[:] * lax.rsqrt(mv_ref[:] + 1e-3)
    y = (y - mm_ref[:]) * inv + be_ref[:]
    d = jnp.sqrt(jnp.maximum(jnp.sum(y * y, axis=-1, keepdims=True), 1e-12))
    o_ref[:] = y / d


@functools.lru_cache(maxsize=None)
def _make_dense(B, H, F):
    BLK = 2048
    assert B % BLK == 0
    vec_spec = pl.BlockSpec((1, F), lambda i: (0, 0))
    return pl.pallas_call(
        functools.partial(_dense_body, H=H),
        grid=(B // BLK,),
        in_specs=[pl.BlockSpec((BLK, F), lambda i: (i, 0))] + [vec_spec] * 5
        + [pl.BlockSpec((F, F), lambda i: (0, 0))],
        out_specs=pl.BlockSpec((BLK, F), lambda i: (i, 0)),
        out_shape=jax.ShapeDtypeStruct((B, F), jnp.float32),
    )


def kernel(x, table, W, b, gamma, beta, moving_mean, moving_var):
    B, H = x.shape
    V, F = table.shape
    xf = jnp.reshape(x.astype(jnp.int32), (B * H,))
    packed = _make_xpose(V, F)(table.T)
    table_rm = jnp.reshape(packed, (packed.shape[0] * 2, F))
    pooled = _make_pool(B, H, V, F)(xf, table_rm)
    dense = _make_dense(B, H, F)
    row = lambda v: jnp.reshape(v, (1, F))
    return dense(pooled, row(b), row(gamma), row(beta), row(moving_mean),
                 row(moving_var), W)


# speedup vs baseline: 1.0473x; 1.0007x over previous
import functools

import jax
import jax.numpy as jnp
from jax import lax
from jax.experimental import pallas as pl
from jax.experimental.pallas import tpu as pltpu
from jax.experimental.pallas import tpu_sc as plsc

_LANES = 16
_MAX_IDX = 128


@functools.lru_cache(maxsize=None)
def _make_pool(B, H, V, F):
    info = plsc.get_sparse_core_info()
    NC, NS = info.num_cores, info.num_subcores
    NW = NC * NS
    assert B % NW == 0
    rows_per_w = B // NW
    C = 32
    assert rows_per_w % C == 0
    nchunk = rows_per_w // C
    idx_per_chunk = C * H
    assert idx_per_chunk % _MAX_IDX == 0
    ng = idx_per_chunk // _MAX_IDX
    nvec = F // _LANES

    mesh = plsc.VectorSubcoreMesh(core_axis_name="c", subcore_axis_name="s")

    @functools.partial(
        pl.kernel,
        mesh=mesh,
        compiler_params=pltpu.CompilerParams(use_tc_tiling_on_sc=False),
        out_type=jax.ShapeDtypeStruct((B, F), jnp.float32),
        scratch_types=[
            pltpu.VMEM((idx_per_chunk,), jnp.int32),
            pltpu.VMEM((idx_per_chunk,), jnp.int32),
            pltpu.VMEM((idx_per_chunk,), jnp.int32),
            pltpu.VMEM((idx_per_chunk, F), jnp.float32),
            pltpu.VMEM((idx_per_chunk, F), jnp.float32),
            pltpu.VMEM((idx_per_chunk, F), jnp.float32),
            pltpu.VMEM((C, F), jnp.float32),
            pltpu.VMEM((C, F), jnp.float32),
            pltpu.SemaphoreType.DMA,
            pltpu.SemaphoreType.DMA,
            pltpu.SemaphoreType.DMA,
            pltpu.SemaphoreType.DMA,
            pltpu.SemaphoreType.DMA,
        ],
    )
    def pool(xf, table, out, idx0, idx1, idx2, rows0, rows1, rows2,
             outc0, outc1, sem0, sem1, sem2, osem0, osem1):
        wid = lax.axis_index("s") * NC + lax.axis_index("c")
        idxs, rows, outcs = [idx0, idx1, idx2], [rows0, rows1, rows2], [outc0, outc1]
        sems, osems = [sem0, sem1, sem2], [osem0, osem1]
        NB = 3

        def start(c):
            b = c % NB
            row0 = wid * rows_per_w + c * C
            pltpu.sync_copy(xf.at[pl.ds(row0 * H, idx_per_chunk)], idxs[b])
            for g in range(idx_per_chunk // _LANES):
                v = idxs[b][pl.ds(g * _LANES, _LANES)]
                j = ((v & jnp.int32(~(_WB - 1)))
                     + ((v & jnp.int32(_HB - 1)) << 1)
                     + ((v >> (_HB.bit_length() - 1)) & 1))
                idxs[b][pl.ds(g * _LANES, _LANES)] = j
            return [pltpu.async_copy(
                table.at[idxs[b].at[pl.ds(g * _MAX_IDX, _MAX_IDX)]],
                rows[b].at[pl.ds(g * _MAX_IDX, _MAX_IDX), :],
                sems[b],
            ) for g in range(ng)]

        def finish(c, copies, ocopy):
            b = c % NB
            for cp in copies:
                cp.wait()
            if ocopy is not None:
                ocopy.wait()
            rows_v, outc_v = rows[b], outcs[c % 2]

            def row_body(r, _):
                for j in range(nvec):
                    a = rows_v[r * H, pl.ds(j * _LANES, _LANES)]
                    for t in range(1, H):
                        a = a + rows_v[r * H + t, pl.ds(j * _LANES, _LANES)]
                    outc_v[r, pl.ds(j * _LANES, _LANES)] = a
                return 0

            lax.fori_loop(0, C, row_body, 0)
            row0 = wid * rows_per_w + c * C
            return pltpu.async_copy(outc_v, out.at[pl.ds(row0, C), :],
                                    osems[c % 2])

        copies = [start(0), start(1), None]
        ocopies = [None, None]
        for c in range(nchunk):
            if c + 2 < nchunk:
                copies[(c + 2) % NB] = start(c + 2)
            ocopies[c % 2] = finish(c, copies[c % NB], ocopies[c % 2])
        ocopies[(nchunk - 2) % 2].wait()
        ocopies[(nchunk - 1) % 2].wait()

    return pool


_WB = 32768
_HB = _WB // 2


def _xpose_body(t_ref, o_ref, *, F):
    o_ref[:, 0:F] = t_ref[:, 0:_HB].T
    o_ref[:, F:2 * F] = t_ref[:, _HB:_WB].T


@functools.lru_cache(maxsize=None)
def _make_xpose(V, F):
    nblk = pl.cdiv(V, _WB)
    return pl.pallas_call(
        functools.partial(_xpose_body, F=F),
        grid=(nblk,),
        in_specs=[pl.BlockSpec((F, _WB), lambda i: (0, i))],
        out_specs=pl.BlockSpec((_HB, 2 * F), lambda i: (i, 0)),
        out_shape=jax.ShapeDtypeStruct((nblk * _HB, 2 * F), jnp.float32),
        compiler_params=pltpu.CompilerParams(
            dimension_semantics=("parallel",)),
    )


def _dense_body(p_ref, b_ref, g_ref, be_ref, mm_ref, mv_ref, w_ref, o_ref, *, H):
    h = p_ref[:] * (1.0 / H)
    y = jnp.dot(h, w_ref[:], preferred_element_type=jnp.float32,
                precision=lax.Precision.HIGHEST)
    y = y + b_ref[:]
    inv = g_ref[:] * lax.rsqrt(mv_ref[:] + 1e-3)
    y = (y - mm_ref[:]) * inv + be_ref[:]
    d = jnp.sqrt(jnp.maximum(jnp.sum(y * y, axis=-1, keepdims=True), 1e-12))
    o_ref[:] = y / d


@functools.lru_cache(maxsize=None)
def _make_dense(B, H, F):
    BLK = 4096
    assert B % BLK == 0
    vec_spec = pl.BlockSpec((1, F), lambda i: (0, 0))
    return pl.pallas_call(
        functools.partial(_dense_body, H=H),
        grid=(B // BLK,),
        in_specs=[pl.BlockSpec((BLK, F), lambda i: (i, 0))] + [vec_spec] * 5
        + [pl.BlockSpec((F, F), lambda i: (0, 0))],
        out_specs=pl.BlockSpec((BLK, F), lambda i: (i, 0)),
        out_shape=jax.ShapeDtypeStruct((B, F), jnp.float32),
        compiler_params=pltpu.CompilerParams(
            dimension_semantics=("parallel",)),
    )


def kernel(x, table, W, b, gamma, beta, moving_mean, moving_var):
    B, H = x.shape
    V, F = table.shape
    xf = jnp.reshape(x.astype(jnp.int32), (B * H,))
    packed = _make_xpose(V, F)(table.T)
    table_rm = jnp.reshape(packed, (packed.shape[0] * 2, F))
    pooled = _make_pool(B, H, V, F)(xf, table_rm)
    dense = _make_dense(B, H, F)
    row = lambda v: jnp.reshape(v, (1, F))
    return dense(pooled, row(b), row(gamma), row(beta), row(moving_mean),
                 row(moving_var), W)


# speedup vs baseline: 1.0677x; 1.0194x over previous
import functools

import jax
import jax.numpy as jnp
from jax import lax
from jax.experimental import pallas as pl
from jax.experimental.pallas import tpu as pltpu
from jax.experimental.pallas import tpu_sc as plsc

_LANES = 16
_MAX_IDX = 128


@functools.lru_cache(maxsize=None)
def _make_pool(B, H, V, F):
    info = plsc.get_sparse_core_info()
    NC, NS = info.num_cores, info.num_subcores
    NW = NC * NS
    assert B % NW == 0
    rows_per_w = B // NW
    C = 32
    assert rows_per_w % C == 0
    nchunk = rows_per_w // C
    idx_per_chunk = C * H
    assert idx_per_chunk % _MAX_IDX == 0
    ng = idx_per_chunk // _MAX_IDX
    nvec = F // _LANES

    mesh = plsc.VectorSubcoreMesh(core_axis_name="c", subcore_axis_name="s")

    @functools.partial(
        pl.kernel,
        mesh=mesh,
        compiler_params=pltpu.CompilerParams(use_tc_tiling_on_sc=False),
        out_type=jax.ShapeDtypeStruct((B, F), jnp.float32),
        scratch_types=[
            pltpu.VMEM((rows_per_w * H,), jnp.int32),
            pltpu.VMEM((idx_per_chunk, F), jnp.float32),
            pltpu.VMEM((idx_per_chunk, F), jnp.float32),
            pltpu.VMEM((C, F), jnp.float32),
            pltpu.VMEM((C, F), jnp.float32),
            pltpu.SemaphoreType.DMA,
            pltpu.SemaphoreType.DMA,
            pltpu.SemaphoreType.DMA,
            pltpu.SemaphoreType.DMA,
        ],
    )
    def pool(xf, table, out, idx_v, rows0, rows1,
             outc0, outc1, sem0, sem1, osem0, osem1):
        wid = lax.axis_index("s") * NC + lax.axis_index("c")
        rows, outcs = [rows0, rows1], [outc0, outc1]
        sems, osems = [sem0, sem1], [osem0, osem1]
        NB = 2

        pltpu.sync_copy(xf.at[pl.ds(wid * rows_per_w * H, rows_per_w * H)],
                        idx_v)
        def remap_body(g, _):
            v = idx_v[pl.ds(g * _LANES, _LANES)]
            j = ((v & jnp.int32(~(_WB - 1)))
                 + ((v & jnp.int32(_HB - 1)) << 1)
                 + ((v >> (_HB.bit_length() - 1)) & 1))
            idx_v[pl.ds(g * _LANES, _LANES)] = j
            return 0

        lax.fori_loop(0, rows_per_w * H // _LANES, remap_body, 0)

        def start(c):
            b = c % NB
            return [pltpu.async_copy(
                table.at[idx_v.at[pl.ds(c * idx_per_chunk + g * _MAX_IDX,
                                        _MAX_IDX)]],
                rows[b].at[pl.ds(g * _MAX_IDX, _MAX_IDX), :],
                sems[b],
            ) for g in range(ng)]

        def finish(c, copies, ocopy):
            b = c % NB
            for cp in copies:
                cp.wait()
            if ocopy is not None:
                ocopy.wait()
            rows_v, outc_v = rows[b], outcs[c % 2]

            def row_body(r, _):
                for j in range(nvec):
                    a = rows_v[r * H, pl.ds(j * _LANES, _LANES)]
                    for t in range(1, H):
                        a = a + rows_v[r * H + t, pl.ds(j * _LANES, _LANES)]
                    outc_v[r, pl.ds(j * _LANES, _LANES)] = a
                return 0

            lax.fori_loop(0, C, row_body, 0)
            row0 = wid * rows_per_w + c * C
            return pltpu.async_copy(outc_v, out.at[pl.ds(row0, C), :],
                                    osems[c % 2])

        copies = [start(0), None]
        ocopies = [None, None]
        for c in range(nchunk):
            if c + 1 < nchunk:
                copies[(c + 1) % NB] = start(c + 1)
            ocopies[c % 2] = finish(c, copies[c % NB], ocopies[c % 2])
        ocopies[(nchunk - 2) % 2].wait()
        ocopies[(nchunk - 1) % 2].wait()

    return pool


_WB = 32768
_HB = _WB // 2


def _xpose_body(t_ref, o_ref, *, F):
    o_ref[:, 0:F] = t_ref[:, 0:_HB].T
    o_ref[:, F:2 * F] = t_ref[:, _HB:_WB].T


@functools.lru_cache(maxsize=None)
def _make_xpose(V, F):
    nblk = pl.cdiv(V, _WB)
    return pl.pallas_call(
        functools.partial(_xpose_body, F=F),
        grid=(nblk,),
        in_specs=[pl.BlockSpec((F, _WB), lambda i: (0, i))],
        out_specs=pl.BlockSpec((_HB, 2 * F), lambda i: (i, 0)),
        out_shape=jax.ShapeDtypeStruct((nblk * _HB, 2 * F), jnp.float32),
        compiler_params=pltpu.CompilerParams(
            dimension_semantics=("parallel",)),
    )


def _dense_body(p_ref, b_ref, g_ref, be_ref, mm_ref, mv_ref, w_ref, o_ref, *, H):
    h = p_ref[:] * (1.0 / H)
    y = jnp.dot(h, w_ref[:], preferred_element_type=jnp.float32,
                precision=lax.Precision.HIGHEST)
    y = y + b_ref[:]
    inv = g_ref[:] * lax.rsqrt(mv_ref[:] + 1e-3)
    y = (y - mm_ref[:]) * inv + be_ref[:]
    d = jnp.sqrt(jnp.maximum(jnp.sum(y * y, axis=-1, keepdims=True), 1e-12))
    o_ref[:] = y / d


@functools.lru_cache(maxsize=None)
def _make_dense(B, H, F):
    BLK = 4096
    assert B % BLK == 0
    vec_spec = pl.BlockSpec((1, F), lambda i: (0, 0))
    return pl.pallas_call(
        functools.partial(_dense_body, H=H),
        grid=(B // BLK,),
        in_specs=[pl.BlockSpec((BLK, F), lambda i: (i, 0))] + [vec_spec] * 5
        + [pl.BlockSpec((F, F), lambda i: (0, 0))],
        out_specs=pl.BlockSpec((BLK, F), lambda i: (i, 0)),
        out_shape=jax.ShapeDtypeStruct((B, F), jnp.float32),
        compiler_params=pltpu.CompilerParams(
            dimension_semantics=("parallel",)),
    )


def kernel(x, table, W, b, gamma, beta, moving_mean, moving_var):
    B, H = x.shape
    V, F = table.shape
    xf = jnp.reshape(x.astype(jnp.int32), (B * H,))
    packed = _make_xpose(V, F)(table.T)
    table_rm = jnp.reshape(packed, (packed.shape[0] * 2, F))
    pooled = _make_pool(B, H, V, F)(xf, table_rm)
    dense = _make_dense(B, H, F)
    row = lambda v: jnp.reshape(v, (1, F))
    return dense(pooled, row(b), row(gamma), row(beta), row(moving_mean),
                 row(moving_var), W)
